# baseline (speedup 1.0000x reference)
import functools

import jax
import jax.numpy as jnp
from jax import lax
from jax.experimental import pallas as pl
from jax.experimental.pallas import tpu as pltpu
from jax.experimental.pallas import tpu_sc as plsc

BATCH = 4096
HIST_LEN = 200
EMBEDDING_DIM = 64
B_TOTAL = BATCH * HIST_LEN

NUM_CORES = 2
NUM_SUBCORES = 16
NUM_WORKERS = NUM_CORES * NUM_SUBCORES
BW = BATCH // NUM_WORKERS
LANES = 16

_mesh = plsc.VectorSubcoreMesh(core_axis_name="c", subcore_axis_name="s")


@functools.partial(
    pl.kernel,
    out_type=jax.ShapeDtypeStruct(
        (HIST_LEN, EMBEDDING_DIM // 8, NUM_WORKERS, 8, BW), jnp.float32
    ),
    mesh=_mesh,
    compiler_params=pltpu.CompilerParams(
        use_tc_tiling_on_sc=False,
        disable_bounds_checks=True,
        needs_layout_passes=False,
    ),
    scratch_types=[
        pltpu.VMEM((HIST_LEN, BW), jnp.int32),
        [pltpu.VMEM((BW, EMBEDDING_DIM), jnp.float32) for _ in range(2)],
        [pltpu.VMEM((EMBEDDING_DIM // 8, 8, BW), jnp.float32) for _ in range(2)],
        [pltpu.SemaphoreType.DMA for _ in range(2)],
        [pltpu.SemaphoreType.DMA for _ in range(2)],
    ],
)
def _sc_embed(idx_hbm, table_hbm, out_hbm, idx_v, rows, blks, gsem, wsem):
    wid = lax.axis_index("s") * NUM_CORES + lax.axis_index("c")
    pltpu.sync_copy(idx_hbm.at[:, pl.ds(wid * BW, BW)], idx_v)

    def gather_copy(t, b):
        return pltpu.make_async_copy(
            table_hbm.at[idx_v.at[t]], rows[b], gsem[b]
        )

    def write_copy(t, b):
        return pltpu.make_async_copy(blks[b], out_hbm.at[t, :, wid], wsem[b])

    def transpose(b):
        def cbody(c, carry):
            ridx = c * LANES + lax.iota(jnp.int32, LANES)
            for d in range(EMBEDDING_DIM):
                v = plsc.load_gather(
                    rows[b], [ridx, jnp.full((LANES,), d, jnp.int32)]
                )
                blks[b][d // 8, d % 8, pl.ds(c * LANES, LANES)] = v
            return carry

        lax.fori_loop(0, BW // LANES, cbody, 0)

    gather_copy(0, 0).start()
    gather_copy(1, 1).start()

    def tbody(g, carry):
        for bs in range(2):
            t = g * 2 + bs
            gather_copy(t, bs).wait()

            @pl.when(t >= 2)
            def _():
                write_copy(t - 2, bs).wait()

            transpose(bs)
            write_copy(t, bs).start()

            @pl.when(t + 2 < HIST_LEN)
            def _():
                gather_copy(t + 2, bs).start()

        return carry

    lax.fori_loop(0, HIST_LEN // 2, tbody, 0)
    write_copy(HIST_LEN - 2, 0).wait()
    write_copy(HIST_LEN - 1, 1).wait()


def kernel(token_ids, weights):
    idx_t = token_ids.T
    out5 = _sc_embed(idx_t, weights)
    return out5.transpose(2, 4, 0, 1, 3).reshape(BATCH, HIST_LEN, EMBEDDING_DIM)


# speedup vs baseline: 1.8529x; 1.8529x over previous
import functools

import jax
import jax.numpy as jnp
from jax import lax
from jax.experimental import pallas as pl
from jax.experimental.pallas import tpu as pltpu
from jax.experimental.pallas import tpu_sc as plsc

BATCH = 4096
HIST_LEN = 200
EMBEDDING_DIM = 64
B_TOTAL = BATCH * HIST_LEN

NUM_CORES = 2
NUM_SUBCORES = 16
NUM_WORKERS = NUM_CORES * NUM_SUBCORES
BW = BATCH // NUM_WORKERS
LANES = 16

_mesh = plsc.VectorSubcoreMesh(core_axis_name="c", subcore_axis_name="s")


@functools.partial(
    pl.kernel,
    out_type=jax.ShapeDtypeStruct(
        (HIST_LEN, EMBEDDING_DIM // 8, NUM_WORKERS, 8, BW), jnp.float32
    ),
    mesh=_mesh,
    compiler_params=pltpu.CompilerParams(
        use_tc_tiling_on_sc=False,
        disable_bounds_checks=True,
        needs_layout_passes=False,
    ),
    scratch_types=[
        pltpu.VMEM((HIST_LEN, BW), jnp.int32),
        [pltpu.VMEM((BW, EMBEDDING_DIM), jnp.float32) for _ in range(2)],
        [pltpu.VMEM((EMBEDDING_DIM // 8, 8, BW + 1), jnp.float32) for _ in range(2)],
        [pltpu.SemaphoreType.DMA for _ in range(2)],
        [pltpu.SemaphoreType.DMA for _ in range(2)],
    ],
)
def _sc_embed(idx_hbm, table_hbm, out_hbm, idx_v, rows, blks, gsem, wsem):
    wid = lax.axis_index("s") * NUM_CORES + lax.axis_index("c")
    pltpu.sync_copy(idx_hbm.at[:, pl.ds(wid * BW, BW)], idx_v)

    def gather_copy(t, b):
        return pltpu.make_async_copy(
            table_hbm.at[idx_v.at[t]], rows[b], gsem[b]
        )

    def write_copy(t, b):
        return pltpu.make_async_copy(
            blks[b].at[:, :, pl.ds(0, BW)], out_hbm.at[t, :, wid], wsem[b]
        )

    _iota = lax.iota(jnp.int32, LANES)
    RUNROLL = 8

    def transpose(b):
        def rbody(r0, carry):
            for ru in range(RUNROLL):
                c = r0 * RUNROLL + ru
                cvec = jnp.full((LANES,), c, jnp.int32)
                for d0 in range(EMBEDDING_DIM // LANES):
                    v = rows[b][c, pl.ds(d0 * LANES, LANES)]
                    d = d0 * LANES + _iota
                    plsc.store_scatter(
                        blks[b],
                        [d >> 3, d & 7, cvec],
                        v,
                    )
            return carry

        lax.fori_loop(0, BW // RUNROLL, rbody, 0)

    gather_copy(0, 0).start()
    gather_copy(1, 1).start()

    def tbody(g, carry):
        for bs in range(2):
            t = g * 2 + bs
            gather_copy(t, bs).wait()

            @pl.when(t >= 2)
            def _():
                write_copy(t - 2, bs).wait()

            transpose(bs)
            write_copy(t, bs).start()

            @pl.when(t + 2 < HIST_LEN)
            def _():
                gather_copy(t + 2, bs).start()

        return carry

    lax.fori_loop(0, HIST_LEN // 2, tbody, 0)
    write_copy(HIST_LEN - 2, 0).wait()
    write_copy(HIST_LEN - 1, 1).wait()


def kernel(token_ids, weights):
    idx_t = token_ids.T
    out5 = _sc_embed(idx_t, weights)
    return out5.transpose(2, 4, 0, 1, 3).reshape(BATCH, HIST_LEN, EMBEDDING_DIM)
